# baseline (speedup 1.0000x reference)
import functools

import jax
import jax.numpy as jnp
from jax import lax
from jax.experimental import pallas as pl
from jax.experimental.pallas import tpu as pltpu
from jax.experimental.pallas import tpu_sc as plsc

_NK = 8
_NC = 2
_NS = 16
_L = 16
_ROWS_PER_CHUNK = 8
_UNROLL = 8
_TC_ROWS = 5120
_TC_BLOCK_ROWS = 512


def _coeff_body(knots_ref, vals_ref, table_ref, aux_ref):
    kn = [knots_ref[i] for i in range(_NK)]
    h = [kn[i + 1] - kn[i] for i in range(_NK - 1)]
    v = vals_ref[...]
    row = [v[i:i + 1, :] for i in range(_NK)]
    slope = [(row[i + 1] - row[i]) * (1.0 / h[i]) for i in range(_NK - 1)]
    zero = jnp.zeros_like(row[0])
    cp = [None] * _NK
    dp = [None] * _NK
    cp[0] = jnp.float32(0.0)
    dp[0] = zero
    for i in range(1, _NK - 1):
        a = h[i - 1]
        b = 2.0 * (h[i - 1] + h[i])
        m = b - a * cp[i - 1]
        cp[i] = h[i] / m
        d_i = 6.0 * (slope[i] - slope[i - 1])
        dp[i] = (d_i - a * dp[i - 1]) * (1.0 / m)
    cp[_NK - 1] = jnp.float32(0.0)
    dp[_NK - 1] = zero
    M = [None] * _NK
    M[_NK - 1] = dp[_NK - 1]
    for i in range(_NK - 2, -1, -1):
        M[i] = dp[i] - cp[i] * M[i + 1]
    for j in range(_NK - 1):
        t0, t1, hj = kn[j], kn[j + 1], h[j]
        ih = 1.0 / hj
        M0, M1 = M[j], M[j + 1]
        y0, y1 = row[j], row[j + 1]
        c3 = (M1 - M0) * (ih / 6.0)
        c2 = (t1 * M0 - t0 * M1) * (ih * 0.5)
        c1 = ((t0 * t0 * M1 - t1 * t1 * M0) * (ih * 0.5)
              + (y1 - y0) * ih - (M1 - M0) * (hj / 6.0))
        c0 = ((t1 * t1 * t1 * M0 - t0 * t0 * t0 * M1) * (ih / 6.0)
              + (y0 * ih - M0 * (hj / 6.0)) * t1
              - (y1 * ih - M1 * (hj / 6.0)) * t0)
        table_ref[pl.ds(4 * j + 0, 1), :] = c3
        table_ref[pl.ds(4 * j + 1, 1), :] = c2
        table_ref[pl.ds(4 * j + 2, 1), :] = c1
        table_ref[pl.ds(4 * j + 3, 1), :] = c0
    inv_h = (_NK - 1.0) / (kn[_NK - 1] - kn[0])
    aux_ref[pl.ds(0, 1), :] = jnp.full((1, _L), kn[0], jnp.float32)
    aux_ref[pl.ds(1, 1), :] = jnp.full((1, _L), inv_h, jnp.float32)


def _coeff_call(knots, vals_t):
    f = vals_t.shape[1]
    return pl.pallas_call(
        _coeff_body,
        out_shape=(
            jax.ShapeDtypeStruct((4 * (_NK - 1), f), jnp.float32),
            jax.ShapeDtypeStruct((2, _L), jnp.float32),
        ),
        in_specs=[
            pl.BlockSpec(memory_space=pltpu.SMEM),
            pl.BlockSpec(memory_space=pltpu.VMEM),
        ],
        out_specs=(
            pl.BlockSpec(memory_space=pltpu.VMEM),
            pl.BlockSpec(memory_space=pltpu.VMEM),
        ),
    )(knots, vals_t)


def _tc_eval_body(knots_ref, x_ref, tbl_ref, out_ref):
    x = x_ref[...]
    masks = [x >= knots_ref[j] for j in range(1, _NK - 1)]
    tbl = tbl_ref[...]
    coef = []
    for c in range(4):
        acc = tbl[c:c + 1, :]
        for j in range(1, _NK - 1):
            acc = jnp.where(masks[j - 1], tbl[4 * j + c:4 * j + c + 1, :], acc)
        coef.append(acc)
    out_ref[...] = ((coef[0] * x + coef[1]) * x + coef[2]) * x + coef[3]


def _tc_eval(knots, x2, table, t_rows, block_rows):
    n_rows, f = x2.shape
    grid = (t_rows // block_rows,)
    return pl.pallas_call(
        _tc_eval_body,
        grid=grid,
        out_shape=jax.ShapeDtypeStruct((n_rows, f), jnp.float32),
        in_specs=[
            pl.BlockSpec(memory_space=pltpu.SMEM),
            pl.BlockSpec((block_rows, f), lambda i: (i, 0)),
            pl.BlockSpec((4 * (_NK - 1), f), lambda i: (0, 0)),
        ],
        out_specs=pl.BlockSpec((block_rows, f), lambda i: (i, 0)),
    )(knots, x2, table)


def _patch_body(t_rows, s_rows, full_ref, sc_ref, out_ref, sem):
    del full_ref
    copy = pltpu.make_async_copy(sc_ref, out_ref.at[pl.ds(t_rows, s_rows)], sem)
    copy.start()
    copy.wait()


def _patch(full_tc, sc_out, t_rows):
    n_rows, f = full_tc.shape
    s_rows = sc_out.shape[0]
    return pl.pallas_call(
        functools.partial(_patch_body, t_rows, s_rows),
        out_shape=jax.ShapeDtypeStruct((n_rows, f), jnp.float32),
        in_specs=[
            pl.BlockSpec(memory_space=pl.ANY),
            pl.BlockSpec(memory_space=pl.ANY),
        ],
        out_specs=pl.BlockSpec(memory_space=pl.ANY),
        scratch_shapes=[pltpu.SemaphoreType.DMA],
        input_output_aliases={0: 0},
    )(full_tc, sc_out)


@functools.lru_cache(maxsize=None)
def _make_sc_eval(n_rows, f, rows_per_chunk, row_offset, total_rows):
    n_workers = _NC * _NS
    rows_per_w = n_rows // n_workers
    n_chunks = rows_per_w // rows_per_chunk
    assert n_chunks % 2 == 0
    groups_per_row = f // _L
    table_words = 4 * (_NK - 1) * f
    mesh = plsc.VectorSubcoreMesh(core_axis_name="c", subcore_axis_name="s")

    @functools.partial(
        pl.kernel,
        mesh=mesh,
        out_type=jax.ShapeDtypeStruct((n_rows, f), jnp.float32),
        scratch_types=[
            pltpu.VMEM((table_words,), jnp.float32),
            pltpu.VMEM((2, _L), jnp.float32),
            pltpu.VMEM((2, rows_per_chunk, f), jnp.float32),
            pltpu.VMEM((2, rows_per_chunk, f), jnp.float32),
            pltpu.SemaphoreType.DMA,
            pltpu.SemaphoreType.DMA,
            pltpu.SemaphoreType.DMA,
            pltpu.SemaphoreType.DMA,
        ],
        compiler_params=pltpu.CompilerParams(
            needs_layout_passes=False, disable_bounds_checks=True),
    )
    def sc_eval(x_hbm, table_hbm, aux_hbm, out_hbm, table_v, aux_v, inb, outb,
                sem_in0, sem_in1, sem_out0, sem_out1):
        wid = lax.axis_index("s") * _NC + lax.axis_index("c")
        base_row = wid * rows_per_w
        sem_in = (sem_in0, sem_in1)
        sem_out = (sem_out0, sem_out1)

        pltpu.sync_copy(table_hbm, table_v)
        pltpu.sync_copy(aux_hbm, aux_v)
        k0v = aux_v[0]
        invhv = aux_v[1]
        lane = lax.iota(jnp.int32, _L)

        def in_copy(chunk, b):
            r0 = row_offset + base_row + chunk * rows_per_chunk
            return pltpu.make_async_copy(
                x_hbm.at[pl.ds(r0, rows_per_chunk)], inb.at[b], sem_in[b])

        def out_copy(chunk, b):
            r0 = base_row + chunk * rows_per_chunk
            return pltpu.make_async_copy(
                outb.at[b], out_hbm.at[pl.ds(r0, rows_per_chunk)], sem_out[b])

        tv1 = table_v.at[pl.ds(f, table_words - f)]
        tv2 = table_v.at[pl.ds(2 * f, table_words - 2 * f)]
        tv3 = table_v.at[pl.ds(3 * f, table_words - 3 * f)]

        def compute(b):
            def row_body(r, _):
                @plsc.parallel_loop(0, groups_per_row, unroll=_UNROLL)
                def grp_body(g):
                    col = g * _L
                    xg = inb[b, r, pl.ds(col, _L)]
                    iv = ((xg - k0v) * invhv).astype(jnp.int32)
                    iv = jnp.minimum(jnp.maximum(iv, 0), _NK - 2)
                    bidx = iv * (4 * f) + (col + lane)
                    c3 = plsc.load_gather(table_v, [bidx])
                    c2 = plsc.load_gather(tv1, [bidx])
                    c1 = plsc.load_gather(tv2, [bidx])
                    c0 = plsc.load_gather(tv3, [bidx])
                    outb[b, r, pl.ds(col, _L)] = (
                        ((c3 * xg + c2) * xg + c1) * xg + c0)

                return 0

            lax.fori_loop(0, rows_per_chunk, row_body, 0)

        in_copy(0, 0).start()
        in_copy(1, 1).start()

        def chunk_pair(ci2, _):
            for b in range(2):
                chunk = ci2 * 2 + b
                in_copy(chunk, b).wait()

                @pl.when(chunk >= 2)
                def _():
                    out_copy(chunk - 2, b).wait()

                compute(b)
                out_copy(chunk, b).start()

                @pl.when(chunk + 2 < n_chunks)
                def _():
                    in_copy(chunk + 2, b).start()

            return 0

        lax.fori_loop(0, n_chunks // 2, chunk_pair, 0)
        out_copy(n_chunks - 2, 0).wait()
        out_copy(n_chunks - 1, 1).wait()

    return sc_eval


def kernel(x, knots, values):
    f = values.shape[0]
    table, aux = _coeff_call(knots, values.T)
    x2 = x.reshape(-1, f)
    n_rows = x2.shape[0]
    t_rows = _TC_ROWS
    sc_rows = n_rows - t_rows
    sc_out = _make_sc_eval(sc_rows, f, _ROWS_PER_CHUNK, t_rows, n_rows)(
        x2, table.reshape(-1), aux)
    if t_rows == 0:
        return sc_out.reshape(x.shape)
    full_tc = _tc_eval(knots, x2, table, t_rows, _TC_BLOCK_ROWS)
    out = _patch(full_tc, sc_out, t_rows)
    return out.reshape(x.shape)


# speedup vs baseline: 8.6035x; 8.6035x over previous
import functools

import jax
import jax.numpy as jnp
from jax import lax
from jax.experimental import pallas as pl
from jax.experimental.pallas import tpu as pltpu
from jax.experimental.pallas import tpu_sc as plsc

_NK = 8
_NC = 2
_NS = 16
_L = 16
_ROWS_PER_CHUNK = 8
_UNROLL = 8
_TC_ROWS = 5120
_TC_BLOCK_ROWS = 512


def _coeff_body(knots_ref, vals_ref, table_ref, aux_ref):
    kn = [knots_ref[i] for i in range(_NK)]
    h = [kn[i + 1] - kn[i] for i in range(_NK - 1)]
    v = vals_ref[...]
    row = [v[i:i + 1, :] for i in range(_NK)]
    slope = [(row[i + 1] - row[i]) * (1.0 / h[i]) for i in range(_NK - 1)]
    zero = jnp.zeros_like(row[0])
    cp = [None] * _NK
    dp = [None] * _NK
    cp[0] = jnp.float32(0.0)
    dp[0] = zero
    for i in range(1, _NK - 1):
        a = h[i - 1]
        b = 2.0 * (h[i - 1] + h[i])
        m = b - a * cp[i - 1]
        cp[i] = h[i] / m
        d_i = 6.0 * (slope[i] - slope[i - 1])
        dp[i] = (d_i - a * dp[i - 1]) * (1.0 / m)
    cp[_NK - 1] = jnp.float32(0.0)
    dp[_NK - 1] = zero
    M = [None] * _NK
    M[_NK - 1] = dp[_NK - 1]
    for i in range(_NK - 2, -1, -1):
        M[i] = dp[i] - cp[i] * M[i + 1]
    for j in range(_NK - 1):
        t0, t1, hj = kn[j], kn[j + 1], h[j]
        ih = 1.0 / hj
        M0, M1 = M[j], M[j + 1]
        y0, y1 = row[j], row[j + 1]
        c3 = (M1 - M0) * (ih / 6.0)
        c2 = (t1 * M0 - t0 * M1) * (ih * 0.5)
        c1 = ((t0 * t0 * M1 - t1 * t1 * M0) * (ih * 0.5)
              + (y1 - y0) * ih - (M1 - M0) * (hj / 6.0))
        c0 = ((t1 * t1 * t1 * M0 - t0 * t0 * t0 * M1) * (ih / 6.0)
              + (y0 * ih - M0 * (hj / 6.0)) * t1
              - (y1 * ih - M1 * (hj / 6.0)) * t0)
        table_ref[pl.ds(4 * j + 0, 1), :] = c3
        table_ref[pl.ds(4 * j + 1, 1), :] = c2
        table_ref[pl.ds(4 * j + 2, 1), :] = c1
        table_ref[pl.ds(4 * j + 3, 1), :] = c0
    inv_h = (_NK - 1.0) / (kn[_NK - 1] - kn[0])
    aux_ref[pl.ds(0, 1), :] = jnp.full((1, _L), kn[0], jnp.float32)
    aux_ref[pl.ds(1, 1), :] = jnp.full((1, _L), inv_h, jnp.float32)


def _coeff_call(knots, vals_t):
    f = vals_t.shape[1]
    return pl.pallas_call(
        _coeff_body,
        out_shape=(
            jax.ShapeDtypeStruct((4 * (_NK - 1), f), jnp.float32),
            jax.ShapeDtypeStruct((2, _L), jnp.float32),
        ),
        in_specs=[
            pl.BlockSpec(memory_space=pltpu.SMEM),
            pl.BlockSpec(memory_space=pltpu.VMEM),
        ],
        out_specs=(
            pl.BlockSpec(memory_space=pltpu.VMEM),
            pl.BlockSpec(memory_space=pltpu.VMEM),
        ),
    )(knots, vals_t)


def _tc_eval_body(knots_ref, x_ref, tbl_ref, out_ref):
    x = x_ref[...]
    masks = [x >= knots_ref[j] for j in range(1, _NK - 1)]
    tbl = tbl_ref[...]
    coef = []
    for c in range(4):
        acc = tbl[c:c + 1, :]
        for j in range(1, _NK - 1):
            acc = jnp.where(masks[j - 1], tbl[4 * j + c:4 * j + c + 1, :], acc)
        coef.append(acc)
    out_ref[...] = ((coef[0] * x + coef[1]) * x + coef[2]) * x + coef[3]


def _tc_eval(knots, x2, table, t_rows, block_rows):
    n_rows, f = x2.shape
    grid = (t_rows // block_rows,)
    return pl.pallas_call(
        _tc_eval_body,
        grid=grid,
        out_shape=jax.ShapeDtypeStruct((n_rows, f), jnp.float32),
        in_specs=[
            pl.BlockSpec(memory_space=pltpu.SMEM),
            pl.BlockSpec((block_rows, f), lambda i: (i, 0)),
            pl.BlockSpec((4 * (_NK - 1), f), lambda i: (0, 0)),
        ],
        out_specs=pl.BlockSpec((block_rows, f), lambda i: (i, 0)),
    )(knots, x2, table)


def _patch_body(full_ref, sc_ref, out_ref):
    del full_ref
    out_ref[...] = sc_ref[...]


def _patch(full_tc, sc_out, t_rows, block_rows):
    n_rows, f = full_tc.shape
    s_rows = sc_out.shape[0]
    t_blocks = t_rows // block_rows
    return pl.pallas_call(
        _patch_body,
        grid=(s_rows // block_rows,),
        out_shape=jax.ShapeDtypeStruct((n_rows, f), jnp.float32),
        in_specs=[
            pl.BlockSpec(memory_space=pl.ANY),
            pl.BlockSpec((block_rows, f), lambda i: (i, 0)),
        ],
        out_specs=pl.BlockSpec((block_rows, f), lambda i: (t_blocks + i, 0)),
        input_output_aliases={0: 0},
    )(full_tc, sc_out)


@functools.lru_cache(maxsize=None)
def _make_sc_eval(n_rows, f, rows_per_chunk, row_offset, total_rows):
    n_workers = _NC * _NS
    rows_per_w = n_rows // n_workers
    n_chunks = rows_per_w // rows_per_chunk
    assert n_chunks % 2 == 0
    groups_per_row = f // _L
    table_words = 4 * (_NK - 1) * f
    mesh = plsc.VectorSubcoreMesh(core_axis_name="c", subcore_axis_name="s")

    @functools.partial(
        pl.kernel,
        mesh=mesh,
        out_type=jax.ShapeDtypeStruct((n_rows, f), jnp.float32),
        scratch_types=[
            pltpu.VMEM((table_words,), jnp.float32),
            pltpu.VMEM((2, _L), jnp.float32),
            pltpu.VMEM((2, rows_per_chunk, f), jnp.float32),
            pltpu.VMEM((2, rows_per_chunk, f), jnp.float32),
            pltpu.SemaphoreType.DMA,
            pltpu.SemaphoreType.DMA,
            pltpu.SemaphoreType.DMA,
            pltpu.SemaphoreType.DMA,
        ],
        compiler_params=pltpu.CompilerParams(
            needs_layout_passes=False, disable_bounds_checks=True),
    )
    def sc_eval(x_hbm, table_hbm, aux_hbm, out_hbm, table_v, aux_v, inb, outb,
                sem_in0, sem_in1, sem_out0, sem_out1):
        wid = lax.axis_index("s") * _NC + lax.axis_index("c")
        base_row = wid * rows_per_w
        sem_in = (sem_in0, sem_in1)
        sem_out = (sem_out0, sem_out1)

        pltpu.sync_copy(table_hbm, table_v)
        pltpu.sync_copy(aux_hbm, aux_v)
        k0v = aux_v[0]
        invhv = aux_v[1]
        lane = lax.iota(jnp.int32, _L)

        def in_copy(chunk, b):
            r0 = row_offset + base_row + chunk * rows_per_chunk
            return pltpu.make_async_copy(
                x_hbm.at[pl.ds(r0, rows_per_chunk)], inb.at[b], sem_in[b])

        def out_copy(chunk, b):
            r0 = base_row + chunk * rows_per_chunk
            return pltpu.make_async_copy(
                outb.at[b], out_hbm.at[pl.ds(r0, rows_per_chunk)], sem_out[b])

        tv1 = table_v.at[pl.ds(f, table_words - f)]
        tv2 = table_v.at[pl.ds(2 * f, table_words - 2 * f)]
        tv3 = table_v.at[pl.ds(3 * f, table_words - 3 * f)]

        def compute(b):
            def row_body(r, _):
                @plsc.parallel_loop(0, groups_per_row, unroll=_UNROLL)
                def grp_body(g):
                    col = g * _L
                    xg = inb[b, r, pl.ds(col, _L)]
                    iv = ((xg - k0v) * invhv).astype(jnp.int32)
                    iv = jnp.minimum(jnp.maximum(iv, 0), _NK - 2)
                    bidx = iv * (4 * f) + (col + lane)
                    c3 = plsc.load_gather(table_v, [bidx])
                    c2 = plsc.load_gather(tv1, [bidx])
                    c1 = plsc.load_gather(tv2, [bidx])
                    c0 = plsc.load_gather(tv3, [bidx])
                    outb[b, r, pl.ds(col, _L)] = (
                        ((c3 * xg + c2) * xg + c1) * xg + c0)

                return 0

            lax.fori_loop(0, rows_per_chunk, row_body, 0)

        in_copy(0, 0).start()
        in_copy(1, 1).start()

        def chunk_pair(ci2, _):
            for b in range(2):
                chunk = ci2 * 2 + b
                in_copy(chunk, b).wait()

                @pl.when(chunk >= 2)
                def _():
                    out_copy(chunk - 2, b).wait()

                compute(b)
                out_copy(chunk, b).start()

                @pl.when(chunk + 2 < n_chunks)
                def _():
                    in_copy(chunk + 2, b).start()

            return 0

        lax.fori_loop(0, n_chunks // 2, chunk_pair, 0)
        out_copy(n_chunks - 2, 0).wait()
        out_copy(n_chunks - 1, 1).wait()

    return sc_eval


def kernel(x, knots, values):
    f = values.shape[0]
    table, aux = _coeff_call(knots, values.T)
    x2 = x.reshape(-1, f)
    n_rows = x2.shape[0]
    t_rows = _TC_ROWS
    sc_rows = n_rows - t_rows
    sc_out = _make_sc_eval(sc_rows, f, _ROWS_PER_CHUNK, t_rows, n_rows)(
        x2, table.reshape(-1), aux)
    if t_rows == 0:
        return sc_out.reshape(x.shape)
    full_tc = _tc_eval(knots, x2, table, t_rows, _TC_BLOCK_ROWS)
    out = _patch(full_tc, sc_out, t_rows, 256)
    return out.reshape(x.shape)
